# baseline (speedup 1.0000x reference)
import functools

import jax
import jax.numpy as jnp
import numpy as np
from jax.experimental import pallas as pl
from jax.experimental.pallas import tpu as pltpu

IN_CH = 96
K_IM = 48
H = 224
W = 224
R = 56
NH = H // R
KS = 7
PAD = 3
EPS = 1e-5


def _fused_kernel(x_ref, mr_ref, mc_ref, w_ref, b_ref, g_ref, bt_ref,
                  oim_ref, osub_ref, xprev_ref, maps_ref, cs_ref, selv_ref):
    n = pl.program_id(1)

    @pl.when(n == 0)
    def _per_batch_setup():
        zslot = jnp.zeros((4, R, W), jnp.float32)
        maps_ref[:, 0] = zslot
        maps_ref[:, NH + 1] = zslot
        mcol = mc_ref[0]
        mrow = mr_ref[0]
        jidx = jax.lax.broadcasted_iota(jnp.int32, (IN_CH, IN_CH), 1)
        cidx = jax.lax.broadcasted_iota(jnp.int32, (IN_CH, IN_CH), 0)
        beats = (mrow > mcol) | ((mrow == mcol) & (jidx < cidx))
        rank = jnp.sum(beats.astype(jnp.float32), axis=1, keepdims=True)
        selv_ref[...] = jnp.where(rank < K_IM, 1.0, 0.0).astype(jnp.float32)

    @pl.when(n < NH)
    def _reduce():
        s_im = mx_im = s_sub = mx_sub = None
        for c in range(IN_CH):
            si = selv_ref[c, 0]
            xi = x_ref[0, c]
            mi = xi * si
            mo = xi - mi
            if c == 0:
                s_im, mx_im, s_sub, mx_sub = mi, mi, mo, mo
            else:
                s_im = s_im + mi
                mx_im = jnp.maximum(mx_im, mi)
                s_sub = s_sub + mo
                mx_sub = jnp.maximum(mx_sub, mo)
        maps_ref[0, n + 1] = s_im * (1.0 / K_IM)
        maps_ref[1, n + 1] = s_sub * (1.0 / (IN_CH - K_IM))
        maps_ref[2, n + 1] = mx_im
        maps_ref[3, n + 1] = mx_sub

    @pl.when(n > 0)
    def _apply():
        top = maps_ref[:, n - 1, R - PAD:R, :]
        mid = maps_ref[:, n]
        bot = maps_ref[:, n + 1, 0:PAD, :]
        wu = jnp.concatenate([top, mid, bot], axis=1)
        for kw in range(KS):
            d = kw - PAD
            if d < 0:
                sh = jnp.concatenate(
                    [jnp.zeros((4, R + 2 * PAD, -d), jnp.float32),
                     wu[:, :, :W + d]], axis=2)
            elif d > 0:
                sh = jnp.concatenate(
                    [wu[:, :, d:],
                     jnp.zeros((4, R + 2 * PAD, d), jnp.float32)], axis=2)
            else:
                sh = wu
            cs_ref[kw] = sh

        scale = g_ref[0] * np.float32(1.0 / np.sqrt(1.0 + EPS))
        bias = b_ref[0]
        beta = bt_ref[0]
        acc = jnp.zeros((2, R, W), jnp.float32)
        for ci in range(2):
            for kh in range(KS):
                for kw in range(KS):
                    acc += w_ref[0, ci, kh, kw] * cs_ref[kw, 2 * ci:2 * ci + 2,
                                                         kh:kh + R, :]
        h = (acc + bias) * scale + beta
        att = jax.nn.sigmoid(jax.nn.relu(h))
        att_im = att[0]
        att_sub = att[1]

        for c in range(IN_CH):
            si = selv_ref[c, 0]
            xi = xprev_ref[c]
            mi = xi * si
            oim_ref[0, c] = mi * att_im
            osub_ref[0, c] = (xi - mi) * att_sub

    @pl.when(n < NH)
    def _retain_x():
        xprev_ref[...] = x_ref[0]


@jax.jit
def kernel(x, M, conv_w, conv_b, bn_gamma, bn_beta):
    B = x.shape[0]
    mrow = M.reshape(B, 1, IN_CH)
    mcol = M.reshape(B, IN_CH, 1)
    f32 = jnp.float32

    out_im, out_sub = pl.pallas_call(
        _fused_kernel,
        grid=(B, NH + 1),
        in_specs=[
            pl.BlockSpec((1, IN_CH, R, W),
                         lambda b, n: (b, 0, jnp.minimum(n, NH - 1), 0)),
            pl.BlockSpec((1, 1, IN_CH), lambda b, n: (b, 0, 0)),
            pl.BlockSpec((1, IN_CH, 1), lambda b, n: (b, 0, 0)),
            pl.BlockSpec((1, 2, KS, KS), lambda b, n: (0, 0, 0, 0)),
            pl.BlockSpec(memory_space=pltpu.SMEM),
            pl.BlockSpec(memory_space=pltpu.SMEM),
            pl.BlockSpec(memory_space=pltpu.SMEM),
        ],
        out_specs=[
            pl.BlockSpec((1, IN_CH, R, W),
                         lambda b, n: (b, 0, jnp.maximum(n - 1, 0), 0)),
            pl.BlockSpec((1, IN_CH, R, W),
                         lambda b, n: (b, 0, jnp.maximum(n - 1, 0), 0)),
        ],
        out_shape=[
            jax.ShapeDtypeStruct((B, IN_CH, H, W), f32),
            jax.ShapeDtypeStruct((B, IN_CH, H, W), f32),
        ],
        scratch_shapes=[
            pltpu.VMEM((IN_CH, R, W), f32),
            pltpu.VMEM((4, NH + 2, R, W), f32),
            pltpu.VMEM((KS, 4, R + 2 * PAD, W), f32),
            pltpu.VMEM((IN_CH, 1), f32),
        ],
    )(x, mrow, mcol, conv_w, conv_b, bn_gamma, bn_beta)

    return (out_im, out_sub)


# speedup vs baseline: 1.0300x; 1.0300x over previous
import functools

import jax
import jax.numpy as jnp
import numpy as np
from jax.experimental import pallas as pl
from jax.experimental.pallas import tpu as pltpu

IN_CH = 96
K_IM = 48
H = 224
W = 224
R = 32
NH = H // R
KS = 7
PAD = 3
EPS = 1e-5


def _fused_kernel(x_ref, mr_ref, mc_ref, w_ref, b_ref, g_ref, bt_ref,
                  oim_ref, osub_ref, xprev_ref, maps_ref, cs_ref, selv_ref):
    n = pl.program_id(1)

    @pl.when(n == 0)
    def _per_batch_setup():
        zslot = jnp.zeros((4, R, W), jnp.float32)
        maps_ref[:, 0] = zslot
        maps_ref[:, NH + 1] = zslot
        mcol = mc_ref[0]
        mrow = mr_ref[0]
        jidx = jax.lax.broadcasted_iota(jnp.int32, (IN_CH, IN_CH), 1)
        cidx = jax.lax.broadcasted_iota(jnp.int32, (IN_CH, IN_CH), 0)
        beats = (mrow > mcol) | ((mrow == mcol) & (jidx < cidx))
        rank = jnp.sum(beats.astype(jnp.float32), axis=1, keepdims=True)
        selv_ref[...] = jnp.where(rank < K_IM, 1.0, 0.0).astype(jnp.float32)

    @pl.when(n < NH)
    def _reduce():
        s_im = mx_im = s_sub = mx_sub = None
        for c in range(IN_CH):
            si = selv_ref[c, 0]
            xi = x_ref[0, c]
            mi = xi * si
            mo = xi - mi
            if c == 0:
                s_im, mx_im, s_sub, mx_sub = mi, mi, mo, mo
            else:
                s_im = s_im + mi
                mx_im = jnp.maximum(mx_im, mi)
                s_sub = s_sub + mo
                mx_sub = jnp.maximum(mx_sub, mo)
        maps_ref[0, n + 1] = s_im * (1.0 / K_IM)
        maps_ref[1, n + 1] = s_sub * (1.0 / (IN_CH - K_IM))
        maps_ref[2, n + 1] = mx_im
        maps_ref[3, n + 1] = mx_sub

    @pl.when(n > 0)
    def _apply():
        top = maps_ref[:, n - 1, R - PAD:R, :]
        mid = maps_ref[:, n]
        bot = maps_ref[:, n + 1, 0:PAD, :]
        wu = jnp.concatenate([top, mid, bot], axis=1)
        for kw in range(KS):
            d = kw - PAD
            if d < 0:
                sh = jnp.concatenate(
                    [jnp.zeros((4, R + 2 * PAD, -d), jnp.float32),
                     wu[:, :, :W + d]], axis=2)
            elif d > 0:
                sh = jnp.concatenate(
                    [wu[:, :, d:],
                     jnp.zeros((4, R + 2 * PAD, d), jnp.float32)], axis=2)
            else:
                sh = wu
            cs_ref[kw] = sh

        scale = g_ref[0] * np.float32(1.0 / np.sqrt(1.0 + EPS))
        bias = b_ref[0]
        beta = bt_ref[0]
        acc = jnp.zeros((2, R, W), jnp.float32)
        for ci in range(2):
            for kh in range(KS):
                for kw in range(KS):
                    acc += w_ref[0, ci, kh, kw] * cs_ref[kw, 2 * ci:2 * ci + 2,
                                                         kh:kh + R, :]
        h = (acc + bias) * scale + beta
        att = jax.nn.sigmoid(jax.nn.relu(h))
        att_im = att[0]
        att_sub = att[1]

        for c in range(IN_CH):
            si = selv_ref[c, 0]
            xi = xprev_ref[c]
            mi = xi * si
            oim_ref[0, c] = mi * att_im
            osub_ref[0, c] = (xi - mi) * att_sub

    @pl.when(n < NH)
    def _retain_x():
        xprev_ref[...] = x_ref[0]


@jax.jit
def kernel(x, M, conv_w, conv_b, bn_gamma, bn_beta):
    B = x.shape[0]
    mrow = M.reshape(B, 1, IN_CH)
    mcol = M.reshape(B, IN_CH, 1)
    f32 = jnp.float32

    out_im, out_sub = pl.pallas_call(
        _fused_kernel,
        grid=(B, NH + 1),
        in_specs=[
            pl.BlockSpec((1, IN_CH, R, W),
                         lambda b, n: (b, 0, jnp.minimum(n, NH - 1), 0)),
            pl.BlockSpec((1, 1, IN_CH), lambda b, n: (b, 0, 0)),
            pl.BlockSpec((1, IN_CH, 1), lambda b, n: (b, 0, 0)),
            pl.BlockSpec((1, 2, KS, KS), lambda b, n: (0, 0, 0, 0)),
            pl.BlockSpec(memory_space=pltpu.SMEM),
            pl.BlockSpec(memory_space=pltpu.SMEM),
            pl.BlockSpec(memory_space=pltpu.SMEM),
        ],
        out_specs=[
            pl.BlockSpec((1, IN_CH, R, W),
                         lambda b, n: (b, 0, jnp.maximum(n - 1, 0), 0)),
            pl.BlockSpec((1, IN_CH, R, W),
                         lambda b, n: (b, 0, jnp.maximum(n - 1, 0), 0)),
        ],
        out_shape=[
            jax.ShapeDtypeStruct((B, IN_CH, H, W), f32),
            jax.ShapeDtypeStruct((B, IN_CH, H, W), f32),
        ],
        scratch_shapes=[
            pltpu.VMEM((IN_CH, R, W), f32),
            pltpu.VMEM((4, NH + 2, R, W), f32),
            pltpu.VMEM((KS, 4, R + 2 * PAD, W), f32),
            pltpu.VMEM((IN_CH, 1), f32),
        ],
    )(x, mrow, mcol, conv_w, conv_b, bn_gamma, bn_beta)

    return (out_im, out_sub)


# speedup vs baseline: 1.0333x; 1.0032x over previous
import functools

import jax
import jax.numpy as jnp
import numpy as np
from jax.experimental import pallas as pl
from jax.experimental.pallas import tpu as pltpu

IN_CH = 96
K_IM = 48
H = 224
W = 224
R = 32
NH = H // R
KS = 7
PAD = 3
EPS = 1e-5


def _fused_kernel(x_ref, mr_ref, mc_ref, w_ref, b_ref, g_ref, bt_ref,
                  oim_ref, osub_ref, xprev_ref, maps_ref, cs_ref, selv_ref):
    n = pl.program_id(1)

    @pl.when(n == 0)
    def _per_batch_setup():
        zslot = jnp.zeros((4, R, W), jnp.float32)
        maps_ref[:, 0] = zslot
        maps_ref[:, NH + 1] = zslot
        mcol = mc_ref[0]
        mrow = mr_ref[0]
        jidx = jax.lax.broadcasted_iota(jnp.int32, (IN_CH, IN_CH), 1)
        cidx = jax.lax.broadcasted_iota(jnp.int32, (IN_CH, IN_CH), 0)
        beats = (mrow > mcol) | ((mrow == mcol) & (jidx < cidx))
        rank = jnp.sum(beats.astype(jnp.float32), axis=1, keepdims=True)
        selv_ref[...] = jnp.where(rank < K_IM, 1.0, 0.0).astype(jnp.float32)

    @pl.when(n < NH)
    def _reduce():
        s_im = mx_im = s_sub = mx_sub = None
        for c in range(IN_CH):
            si = selv_ref[c, 0]
            xi = x_ref[0, c]
            mi = xi * si
            mo = xi - mi
            if c == 0:
                s_im, mx_im, s_sub, mx_sub = mi, mi, mo, mo
            else:
                s_im = s_im + mi
                mx_im = jnp.maximum(mx_im, mi)
                s_sub = s_sub + mo
                mx_sub = jnp.maximum(mx_sub, mo)
        maps_ref[0, n + 1] = s_im * (1.0 / K_IM)
        maps_ref[1, n + 1] = s_sub * (1.0 / (IN_CH - K_IM))
        maps_ref[2, n + 1] = mx_im
        maps_ref[3, n + 1] = mx_sub

    @pl.when(n > 0)
    def _apply():
        top = maps_ref[:, n - 1, R - PAD:R, :]
        mid = maps_ref[:, n]
        bot = maps_ref[:, n + 1, 0:PAD, :]
        wu = jnp.concatenate([top, mid, bot], axis=1)
        for kw in range(KS):
            d = kw - PAD
            if d < 0:
                sh = jnp.concatenate(
                    [jnp.zeros((4, R + 2 * PAD, -d), jnp.float32),
                     wu[:, :, :W + d]], axis=2)
            elif d > 0:
                sh = jnp.concatenate(
                    [wu[:, :, d:],
                     jnp.zeros((4, R + 2 * PAD, d), jnp.float32)], axis=2)
            else:
                sh = wu
            cs_ref[kw] = sh

        scale = g_ref[0] * np.float32(1.0 / np.sqrt(1.0 + EPS))
        bias = b_ref[0]
        beta = bt_ref[0]
        acc = jnp.zeros((2, R, W), jnp.float32)
        for ci in range(2):
            for kh in range(KS):
                for kw in range(KS):
                    acc += w_ref[0, ci, kh, kw] * cs_ref[kw, 2 * ci:2 * ci + 2,
                                                         kh:kh + R, :]
        h = (acc + bias) * scale + beta
        att = jax.nn.sigmoid(jax.nn.relu(h))
        att_im = att[0]
        att_sub = att[1]

        for c in range(IN_CH):
            si = selv_ref[c, 0]
            xi = xprev_ref[c]
            mi = xi * si
            oim_ref[0, c] = mi * att_im
            osub_ref[0, c] = (xi - mi) * att_sub

    @pl.when(n < NH)
    def _retain_x():
        xprev_ref[...] = x_ref[0]


@jax.jit
def kernel(x, M, conv_w, conv_b, bn_gamma, bn_beta):
    B = x.shape[0]
    mrow = M.reshape(B, 1, IN_CH)
    mcol = M.reshape(B, IN_CH, 1)
    f32 = jnp.float32

    out_im, out_sub = pl.pallas_call(
        _fused_kernel,
        grid=(B, NH + 1),
        compiler_params=pltpu.CompilerParams(
            dimension_semantics=("parallel", "arbitrary")),
        in_specs=[
            pl.BlockSpec((1, IN_CH, R, W),
                         lambda b, n: (b, 0, jnp.minimum(n, NH - 1), 0)),
            pl.BlockSpec((1, 1, IN_CH), lambda b, n: (b, 0, 0)),
            pl.BlockSpec((1, IN_CH, 1), lambda b, n: (b, 0, 0)),
            pl.BlockSpec((1, 2, KS, KS), lambda b, n: (0, 0, 0, 0)),
            pl.BlockSpec(memory_space=pltpu.SMEM),
            pl.BlockSpec(memory_space=pltpu.SMEM),
            pl.BlockSpec(memory_space=pltpu.SMEM),
        ],
        out_specs=[
            pl.BlockSpec((1, IN_CH, R, W),
                         lambda b, n: (b, 0, jnp.maximum(n - 1, 0), 0)),
            pl.BlockSpec((1, IN_CH, R, W),
                         lambda b, n: (b, 0, jnp.maximum(n - 1, 0), 0)),
        ],
        out_shape=[
            jax.ShapeDtypeStruct((B, IN_CH, H, W), f32),
            jax.ShapeDtypeStruct((B, IN_CH, H, W), f32),
        ],
        scratch_shapes=[
            pltpu.VMEM((IN_CH, R, W), f32),
            pltpu.VMEM((4, NH + 2, R, W), f32),
            pltpu.VMEM((KS, 4, R + 2 * PAD, W), f32),
            pltpu.VMEM((IN_CH, 1), f32),
        ],
    )(x, mrow, mcol, conv_w, conv_b, bn_gamma, bn_beta)

    return (out_im, out_sub)
